# baseline (device time: 20250 ns/iter reference)
import jax
import jax.numpy as jnp
from jax import lax
from jax.experimental import pallas as pl
from jax.experimental.pallas import tpu as pltpu

N_DEV = 32


def kernel(x, w_mat):
    m, k_per = x.shape
    k, n = w_mat.shape
    m_out = m // N_DEV

    def body(x_ref, w_ref, out_ref, comm_ref, send_sems, recv_sems):
        my = lax.axis_index("i")

        barrier = pltpu.get_barrier_semaphore()
        for d in range(1, N_DEV):
            peer = lax.rem(my + d, N_DEV)
            pl.semaphore_signal(
                barrier, inc=1,
                device_id=(peer,), device_id_type=pl.DeviceIdType.MESH,
            )
        pl.semaphore_wait(barrier, N_DEV - 1)

        rdmas = []
        for d in range(1, N_DEV):
            tgt = lax.rem(my + d, N_DEV)
            rdma = pltpu.make_async_remote_copy(
                src_ref=x_ref.at[pl.ds(tgt * m_out, m_out), :],
                dst_ref=comm_ref.at[d],
                send_sem=send_sems.at[d],
                recv_sem=recv_sems.at[d],
                device_id=(tgt,),
                device_id_type=pl.DeviceIdType.MESH,
            )
            rdma.start()
            rdmas.append(rdma)

        acc = jnp.dot(
            x_ref[pl.ds(my * m_out, m_out), :],
            w_ref[pl.ds(my * k_per, k_per), :],
            preferred_element_type=jnp.float32,
        )
        for d in range(1, N_DEV):
            rdmas[d - 1].wait_recv()
            src = lax.rem(my + (N_DEV - d), N_DEV)
            acc = acc + jnp.dot(
                comm_ref[d],
                w_ref[pl.ds(src * k_per, k_per), :],
                preferred_element_type=jnp.float32,
            )
        for d in range(1, N_DEV):
            rdmas[d - 1].wait_send()

        out_ref[:, :] = acc * jax.nn.sigmoid(acc)

    return pl.pallas_call(
        body,
        out_shape=jax.ShapeDtypeStruct((m_out, n), jnp.float32),
        in_specs=[
            pl.BlockSpec(memory_space=pltpu.VMEM),
            pl.BlockSpec(memory_space=pltpu.VMEM),
        ],
        out_specs=pl.BlockSpec(memory_space=pltpu.VMEM),
        scratch_shapes=[
            pltpu.VMEM((N_DEV, m_out, k_per), x.dtype),
            pltpu.SemaphoreType.DMA((N_DEV,)),
            pltpu.SemaphoreType.DMA((N_DEV,)),
        ],
        compiler_params=pltpu.CompilerParams(collective_id=0),
    )(x, w_mat)


# device time: 20126 ns/iter; 1.0062x vs baseline; 1.0062x over previous
import jax
import jax.numpy as jnp
from jax import lax
from jax.experimental import pallas as pl
from jax.experimental.pallas import tpu as pltpu

N_DEV = 32


def kernel(x, w_mat):
    m, k_per = x.shape
    k, n = w_mat.shape
    m_out = m // N_DEV

    def body(x_ref, w_ref, out_ref, comm_ref, send_sems, recv_sems):
        my = lax.axis_index("i")

        barrier = pltpu.get_barrier_semaphore()
        for d in range(1, N_DEV):
            peer = lax.rem(my + d, N_DEV)
            pl.semaphore_signal(
                barrier, inc=1,
                device_id=(peer,), device_id_type=pl.DeviceIdType.MESH,
            )
        pl.semaphore_wait(barrier, N_DEV - 1)

        rdmas = []
        for d in range(1, N_DEV):
            tgt = lax.rem(my + d, N_DEV)
            rdma = pltpu.make_async_remote_copy(
                src_ref=x_ref.at[pl.ds(tgt * m_out, m_out), :],
                dst_ref=comm_ref.at[my],
                send_sem=send_sems.at[d],
                recv_sem=recv_sems.at[d],
                device_id=(tgt,),
                device_id_type=pl.DeviceIdType.MESH,
            )
            rdma.start()
            rdmas.append(rdma)

        comm_ref[my] = x_ref[pl.ds(my * m_out, m_out), :]

        for d in range(1, N_DEV):
            rdmas[d - 1].wait_recv()
        for d in range(1, N_DEV):
            rdmas[d - 1].wait_send()

        xrow = jnp.concatenate(
            [comm_ref[j] for j in range(N_DEV)], axis=1
        )
        acc = jnp.dot(xrow, w_ref[:, :], preferred_element_type=jnp.float32)
        out_ref[:, :] = acc * jax.nn.sigmoid(acc)

    return pl.pallas_call(
        body,
        out_shape=jax.ShapeDtypeStruct((m_out, n), jnp.float32),
        in_specs=[
            pl.BlockSpec(memory_space=pltpu.VMEM),
            pl.BlockSpec(memory_space=pltpu.VMEM),
        ],
        out_specs=pl.BlockSpec(memory_space=pltpu.VMEM),
        scratch_shapes=[
            pltpu.VMEM((N_DEV, m_out, k_per), x.dtype),
            pltpu.SemaphoreType.DMA((N_DEV,)),
            pltpu.SemaphoreType.DMA((N_DEV,)),
        ],
        compiler_params=pltpu.CompilerParams(collective_id=0),
    )(x, w_mat)
